# Initial kernel scaffold; baseline (speedup 1.0000x reference)
import jax
import jax.numpy as jnp
from jax.experimental import pallas as pl


def kernel(x, table, W, b):
    raise NotImplementedError("write your pallas kernel here")


# speedup vs baseline: 11.1229x; 11.1229x over previous
import functools

import jax
import jax.numpy as jnp
from jax import lax
from jax.experimental import pallas as pl
from jax.experimental.pallas import tpu as pltpu
from jax.experimental.pallas import tpu_sc as plsc

_VOCAB = 1000000
_CD = 16
_ED = 64
_NB = 16384
_NF = 26
_N = _NB * _NF

_NC = 2
_NS = 16
_NW = _NC * _NS
_PER_W = _N // _NW
_CHUNK = 1664
_NCHUNK = _PER_W // _CHUNK

_PACK = 8
_MM_ROWS = _N // _PACK
_MM_BLK = 2048


def _sc_gather(table, idx):
    mesh = plsc.VectorSubcoreMesh(core_axis_name="c", subcore_axis_name="s")

    @functools.partial(
        pl.kernel,
        mesh=mesh,
        out_type=jax.ShapeDtypeStruct((_N, _CD), jnp.float32),
        compiler_params=pltpu.CompilerParams(use_tc_tiling_on_sc=False),
        scratch_types=[
            pltpu.VMEM((_CHUNK,), jnp.int32),
            pltpu.VMEM((_CHUNK, _CD), jnp.float32),
            pltpu.SemaphoreType.DMA,
        ],
    )
    def k(table_hbm, idx_hbm, out_hbm, idx_v, rows_v, sem):
        wid = lax.axis_index("s") * _NC + lax.axis_index("c")
        base = wid * _PER_W
        for c in range(_NCHUNK):
            off = base + c * _CHUNK
            pltpu.sync_copy(idx_hbm.at[pl.ds(off, _CHUNK)], idx_v)
            pltpu.async_copy(table_hbm.at[idx_v], rows_v, sem).wait()
            pltpu.sync_copy(rows_v, out_hbm.at[pl.ds(off, _CHUNK)])

    return k(table, idx)


def _mm_body(e_ref, w_ref, b_ref, o_ref):
    o_ref[...] = (
        jnp.dot(e_ref[...], w_ref[...], preferred_element_type=jnp.float32)
        + b_ref[...]
    )


def _tc_project(emb_p, big_w, bias_p):
    return pl.pallas_call(
        _mm_body,
        grid=(_MM_ROWS // _MM_BLK,),
        in_specs=[
            pl.BlockSpec((_MM_BLK, _PACK * _CD), lambda i: (i, 0)),
            pl.BlockSpec((_PACK * _CD, _PACK * _ED), lambda i: (0, 0)),
            pl.BlockSpec((1, _PACK * _ED), lambda i: (0, 0)),
        ],
        out_specs=pl.BlockSpec((_MM_BLK, _PACK * _ED), lambda i: (i, 0)),
        out_shape=jax.ShapeDtypeStruct((_MM_ROWS, _PACK * _ED), jnp.float32),
    )(emb_p, big_w, bias_p)


def kernel(x, table, W, b):
    idx = x.reshape(-1).astype(jnp.int32)
    emb = _sc_gather(table, idx)
    emb_p = emb.reshape(_MM_ROWS, _PACK * _CD)
    big_w = jnp.kron(jnp.eye(_PACK, dtype=W.dtype), W.T)
    bias_p = jnp.tile(b, _PACK)[None, :]
    out_p = _tc_project(emb_p, big_w, bias_p)
    return out_p.reshape(_NB, _NF, _ED)
